# Initial kernel scaffold; baseline (speedup 1.0000x reference)
import jax
import jax.numpy as jnp
from jax.experimental import pallas as pl


def kernel(x, w_ij, seg_i, idx_j, seg_i_sum, W_in, W_out, b_out):
    raise NotImplementedError("write your pallas kernel here")


# speedup vs baseline: 7.1971x; 7.1971x over previous
import functools

import jax
import jax.numpy as jnp
from jax import lax
from jax.experimental import pallas as pl
from jax.experimental.pallas import tpu as pltpu
from jax.experimental.pallas import tpu_sc as plsc

N = 10000
E = 320000
F = 128
FH = F // 2
NC = 2
NS = 16
EPT = E // NS
C = 80
NCHT = EPT // C
N_PAD = 10240
ROWS_PT = N_PAD // NS
ZR = 128
LANES = 16

_mesh = plsc.VectorSubcoreMesh(core_axis_name="c", subcore_axis_name="s",
                               num_cores=NC)


@functools.partial(
    pl.kernel,
    out_type=jax.ShapeDtypeStruct((NC, N_PAD, FH), jnp.float32),
    mesh=_mesh,
    compiler_params=pltpu.CompilerParams(use_tc_tiling_on_sc=False),
    scratch_types=[
        pltpu.VMEM((NCHT, C), jnp.int32),
        pltpu.VMEM((NCHT, C), jnp.int32),
        pltpu.VMEM((C, FH), jnp.float32),
        pltpu.VMEM((C, FH), jnp.float32),
        pltpu.VMEM((C, FH), jnp.float32),
        pltpu.VMEM((C, FH), jnp.float32),
        pltpu.VMEM((C,), jnp.int32),
        pltpu.VMEM((C,), jnp.int32),
        pltpu.VMEM((ZR, FH), jnp.float32),
        pltpu.VMEM_SHARED((N_PAD, FH), jnp.float32),
        pltpu.SemaphoreType.DMA,
        pltpu.SemaphoreType.DMA,
        pltpu.SemaphoreType.DMA,
        pltpu.SemaphoreType.DMA,
    ],
)
def _edge_kernel(f2_hbm, w2_hbm, seg_hbm, idx_hbm, out_hbm,
                 idx_v, seg_v, rows0, rows1, w0, w1, wi0, wi1, zbuf, accum,
                 gsem0, gsem1, wsem0, wsem1):
    cid = lax.axis_index("c")
    sid = lax.axis_index("s")

    zero = jnp.zeros((LANES,), jnp.float32)

    def _zero_row(r, _):
        for g in range(FH // LANES):
            zbuf[r, pl.ds(g * LANES, LANES)] = zero
        return 0

    lax.fori_loop(0, ZR, _zero_row, 0)
    base = sid * ROWS_PT
    for k in range(ROWS_PT // ZR):
        pltpu.sync_copy(zbuf, accum.at[pl.ds(base + k * ZR, ZR)])
    plsc.subcore_barrier()

    pltpu.sync_copy(idx_hbm.at[sid], idx_v)
    pltpu.sync_copy(seg_hbm.at[sid], seg_v)

    two_iota = lax.iota(jnp.int32, LANES) * 2

    def _xform(g, _):
        for k in range(C // LANES):
            sl = pl.ds(k * LANES, LANES)
            idx_v[g, sl] = idx_v[g, sl] * 2 + cid
        return 0

    lax.fori_loop(0, NCHT, _xform, 0)

    e_base2 = (sid * EPT) * 2 + cid

    bufs = ((rows0, w0, wi0, gsem0, wsem0), (rows1, w1, wi1, gsem1, wsem1))

    def _start(g, rows, w, wi, gsem, wsem):
        e0 = e_base2 + g * (2 * C)
        for k in range(C // LANES):
            wi[pl.ds(k * LANES, LANES)] = two_iota + (e0 + 2 * k * LANES)
        pltpu.async_copy(f2_hbm.at[idx_v.at[g]], rows, gsem)
        pltpu.async_copy(w2_hbm.at[wi], w, wsem)

    def _mul(rows, w):
        def body(r, _):
            for g in range(FH // LANES):
                sl = pl.ds(g * LANES, LANES)
                rows[r, sl] = rows[r, sl] * w[r, sl]
            return 0
        lax.fori_loop(0, C, body, 0)

    _start(0, *bufs[0])
    _start(1, *bufs[1])

    def loop_body(g2, _):
        g = g2 * 2
        for b in range(2):
            rows, w, wi, gsem, wsem = bufs[b]
            gcur = g + b
            pltpu.make_async_copy(f2_hbm.at[idx_v.at[gcur]], rows, gsem).wait()
            pltpu.make_async_copy(w2_hbm.at[wi], w, wsem).wait()
            _mul(rows, w)
            pltpu.sync_copy(rows, accum.at[seg_v.at[gcur]], add=True)

            @pl.when(gcur + 2 < NCHT)
            def _():
                _start(gcur + 2, rows, w, wi, gsem, wsem)
        return 0

    lax.fori_loop(0, NCHT // 2, loop_body, 0)

    plsc.subcore_barrier()
    pltpu.sync_copy(accum.at[pl.ds(base, ROWS_PT)],
                    out_hbm.at[cid, pl.ds(base, ROWS_PT)])


def _mm_in_body(x_ref, w_ref, o_ref):
    o_ref[...] = jnp.dot(x_ref[...], w_ref[...],
                         preferred_element_type=jnp.float32)


def _mm_out_body(p_ref, w_ref, b_ref, o_ref):
    conv = jnp.concatenate([p_ref[0], p_ref[1]], axis=-1)
    o_ref[...] = jnp.dot(conv, w_ref[...],
                         preferred_element_type=jnp.float32) + b_ref[...]


_BLK = 1000


def kernel(x, w_ij, seg_i, idx_j, seg_i_sum, W_in, W_out, b_out):
    f = pl.pallas_call(
        _mm_in_body,
        grid=(N // _BLK,),
        in_specs=[pl.BlockSpec((_BLK, F), lambda i: (i, 0)),
                  pl.BlockSpec((F, F), lambda i: (0, 0))],
        out_specs=pl.BlockSpec((_BLK, F), lambda i: (i, 0)),
        out_shape=jax.ShapeDtypeStruct((N, F), jnp.float32),
    )(x, W_in)

    f2 = f.reshape(2 * N, FH)
    w2 = w_ij.reshape(2 * E, FH)
    idx3 = idx_j.astype(jnp.int32).reshape(NS, NCHT, C)
    seg3 = seg_i.astype(jnp.int32).reshape(NS, NCHT, C)

    partials = _edge_kernel(f2, w2, seg3, idx3)

    bias = (b_out
            + (jnp.asarray(seg_i_sum, jnp.float32) - jnp.float32(N))
            ).reshape(1, F)

    c = pl.pallas_call(
        _mm_out_body,
        grid=(N // _BLK,),
        in_specs=[pl.BlockSpec((NC, _BLK, FH), lambda i: (0, i, 0)),
                  pl.BlockSpec((F, F), lambda i: (0, 0)),
                  pl.BlockSpec((1, F), lambda i: (0, 0))],
        out_specs=pl.BlockSpec((_BLK, F), lambda i: (i, 0)),
        out_shape=jax.ShapeDtypeStruct((N, F), jnp.float32),
    )(partials, W_out, bias)
    return c


# speedup vs baseline: 8.7034x; 1.2093x over previous
import functools

import jax
import jax.numpy as jnp
from jax import lax
from jax.experimental import pallas as pl
from jax.experimental.pallas import tpu as pltpu
from jax.experimental.pallas import tpu_sc as plsc

N = 10000
E = 320000
F = 128
FH = F // 2
NC = 2
NS = 16
EPT = E // NS
C = 80
NCHT = EPT // C
NBUF = 5
PRE = NBUF - 1
N_PAD = 10240
ROWS_PT = N_PAD // NS
ZR = 64
LANES = 16

_mesh = plsc.VectorSubcoreMesh(core_axis_name="c", subcore_axis_name="s",
                               num_cores=NC)


def _buf_types():
    ts = [pltpu.VMEM((NCHT, C), jnp.int32)]
    for _ in range(NBUF):
        ts += [pltpu.VMEM((C, FH), jnp.float32),
               pltpu.VMEM((C, FH), jnp.float32),
               pltpu.VMEM((C,), jnp.int32),
               pltpu.VMEM((C,), jnp.int32)]
    ts.append(pltpu.VMEM((ZR, FH), jnp.float32))
    ts.append(pltpu.VMEM_SHARED((N_PAD, FH), jnp.float32))
    for _ in range(NBUF):
        ts += [pltpu.SemaphoreType.DMA,
               pltpu.SemaphoreType.DMA,
               pltpu.SemaphoreType.DMA,
               pltpu.SemaphoreType.DMA]
    return ts


@functools.partial(
    pl.kernel,
    out_type=jax.ShapeDtypeStruct((NC, N_PAD, FH), jnp.float32),
    mesh=_mesh,
    compiler_params=pltpu.CompilerParams(use_tc_tiling_on_sc=False),
    scratch_types=_buf_types(),
)
def _edge_kernel(f2_hbm, w2_hbm, seg_hbm, idx_hbm, out_hbm, idx_v, *scratch):
    bufs = tuple(scratch[b * 4:b * 4 + 4] for b in range(NBUF))
    zbuf = scratch[NBUF * 4]
    accum = scratch[NBUF * 4 + 1]
    sems = tuple(scratch[NBUF * 4 + 2 + b * 4:NBUF * 4 + 6 + b * 4]
                 for b in range(NBUF))

    cid = lax.axis_index("c")
    sid = lax.axis_index("s")

    zero = jnp.zeros((LANES,), jnp.float32)

    def _zero_row(r, _):
        for g in range(FH // LANES):
            zbuf[r, pl.ds(g * LANES, LANES)] = zero
        return 0

    lax.fori_loop(0, ZR, _zero_row, 0)
    base = sid * ROWS_PT
    for k in range(ROWS_PT // ZR):
        pltpu.sync_copy(zbuf, accum.at[pl.ds(base + k * ZR, ZR)])
    plsc.subcore_barrier()

    pltpu.sync_copy(idx_hbm.at[sid], idx_v)

    two_iota = lax.iota(jnp.int32, LANES) * 2

    def _xform(g, _):
        for k in range(C // LANES):
            sl = pl.ds(k * LANES, LANES)
            idx_v[g, sl] = idx_v[g, sl] * 2 + cid
        return 0

    lax.fori_loop(0, NCHT, _xform, 0)

    e_base2 = (sid * EPT) * 2 + cid

    def _start(g, b):
        rows, w, wi, segb = bufs[b]
        gsem, wsem, ssem, _ = sems[b]
        e0 = e_base2 + g * (2 * C)
        for k in range(C // LANES):
            wi[pl.ds(k * LANES, LANES)] = two_iota + (e0 + 2 * k * LANES)
        pltpu.async_copy(f2_hbm.at[idx_v.at[g]], rows, gsem)
        pltpu.async_copy(w2_hbm.at[wi], w, wsem)
        pltpu.async_copy(seg_hbm.at[sid, g], segb, ssem)

    UNROLL = 4

    def _mul(rows, w):
        def body(r4, _):
            for u in range(UNROLL):
                for g in range(FH // LANES):
                    sl = pl.ds(g * LANES, LANES)
                    r = r4 * UNROLL + u
                    rows[r, sl] = rows[r, sl] * w[r, sl]
            return 0
        lax.fori_loop(0, C // UNROLL, body, 0)

    for p in range(PRE):
        _start(p, p)

    def outer(q, _):
        for b in range(NBUF):
            g = q * NBUF + b
            rows, w, wi, segb = bufs[b]
            gsem, wsem, ssem, scsem = sems[b]
            pltpu.make_async_copy(f2_hbm.at[idx_v.at[g]], rows, gsem).wait()
            pltpu.make_async_copy(w2_hbm.at[wi], w, wsem).wait()
            pltpu.make_async_copy(seg_hbm.at[sid, g], segb, ssem).wait()
            _mul(rows, w)
            pltpu.async_copy(rows, accum.at[segb], scsem, add=True)

            nb = (b + PRE) % NBUF
            nrows, _, _, nsegb = bufs[nb]
            nscsem = sems[nb][3]

            @pl.when(g + PRE < NCHT)
            def _():
                @pl.when(g > 0)
                def _():
                    pltpu.make_async_copy(
                        nrows, accum.at[nsegb], nscsem).wait()
                _start(g + PRE, nb)
        return 0

    lax.fori_loop(0, NCHT // NBUF, outer, 0)

    for b in range(NBUF):
        rows, _, _, segb = bufs[b]
        scsem = sems[b][3]
        pltpu.make_async_copy(rows, accum.at[segb], scsem).wait()

    plsc.subcore_barrier()
    pltpu.sync_copy(accum.at[pl.ds(base, ROWS_PT)],
                    out_hbm.at[cid, pl.ds(base, ROWS_PT)])


def _mm_in_body(x_ref, w_ref, o_ref):
    o_ref[...] = jnp.dot(x_ref[...], w_ref[...],
                         preferred_element_type=jnp.float32)


def _mm_out_body(p_ref, w_ref, b_ref, o_ref):
    conv = jnp.concatenate([p_ref[0], p_ref[1]], axis=-1)
    o_ref[...] = jnp.dot(conv, w_ref[...],
                         preferred_element_type=jnp.float32) + b_ref[...]


_BLK = 1000


def kernel(x, w_ij, seg_i, idx_j, seg_i_sum, W_in, W_out, b_out):
    f = pl.pallas_call(
        _mm_in_body,
        grid=(N // _BLK,),
        in_specs=[pl.BlockSpec((_BLK, F), lambda i: (i, 0)),
                  pl.BlockSpec((F, F), lambda i: (0, 0))],
        out_specs=pl.BlockSpec((_BLK, F), lambda i: (i, 0)),
        out_shape=jax.ShapeDtypeStruct((N, F), jnp.float32),
    )(x, W_in)

    f2 = f.reshape(2 * N, FH)
    w2 = w_ij.reshape(2 * E, FH)
    idx3 = idx_j.astype(jnp.int32).reshape(NS, NCHT, C)
    seg3 = seg_i.astype(jnp.int32).reshape(NS, NCHT, C)

    partials = _edge_kernel(f2, w2, seg3, idx3)

    bias = (b_out
            + (jnp.asarray(seg_i_sum, jnp.float32) - jnp.float32(N))
            ).reshape(1, F)

    c = pl.pallas_call(
        _mm_out_body,
        grid=(N // _BLK,),
        in_specs=[pl.BlockSpec((NC, _BLK, FH), lambda i: (0, i, 0)),
                  pl.BlockSpec((F, F), lambda i: (0, 0)),
                  pl.BlockSpec((1, F), lambda i: (0, 0))],
        out_specs=pl.BlockSpec((_BLK, F), lambda i: (i, 0)),
        out_shape=jax.ShapeDtypeStruct((N, F), jnp.float32),
    )(partials, W_out, bias)
    return c
